# Initial kernel scaffold; baseline (speedup 1.0000x reference)
import jax
import jax.numpy as jnp
from jax.experimental import pallas as pl


def kernel(hidden_states, W_gate, Wg, Wu, W2):
    raise NotImplementedError("write your pallas kernel here")


# speedup vs baseline: 2.3296x; 2.3296x over previous
import jax
import jax.numpy as jnp
from jax import lax
from jax.experimental import pallas as pl
from jax.experimental.pallas import tpu as pltpu
from jax.experimental.pallas import tpu_sc as plsc

T = 2048
H = 1024
DFF = 2048
K = 2
E = 8
TILE_M = 128
NT = (T * K) // TILE_M + E
P = NT * TILE_M
NC, NS = 2, 16
NW = NC * NS

_f32 = jnp.float32
_bf16 = jnp.bfloat16
_i32 = jnp.int32


def _gate_body(hs_ref, wgate_ref, scores_ref, dest_ref, te_ref, oh_scr, c_scr):
    hs = hs_ref[...]
    logits = lax.dot_general(hs, wgate_ref[...], (((1,), (1,)), ((), ())),
                             preferred_element_type=_f32)
    col = lax.broadcasted_iota(_i32, (T, E), 1)
    a1 = jnp.argmax(logits, axis=1)
    oh0 = col == a1[:, None]
    masked = jnp.where(oh0, -jnp.inf, logits)
    a2 = jnp.argmax(masked, axis=1)
    oh1 = col == a2[:, None]
    m1 = jnp.max(logits, axis=1, keepdims=True)
    m2 = jnp.max(masked, axis=1, keepdims=True)
    e2 = jnp.exp(m2 - m1)
    denom = 1.0 + e2
    scores_ref[:, 0:1] = 1.0 / denom
    scores_ref[:, 1:2] = e2 / denom

    oh_scr[...] = (oh0 | oh1).astype(_bf16)
    tri = (lax.broadcasted_iota(_i32, (TILE_M, TILE_M), 0)
           > lax.broadcasted_iota(_i32, (TILE_M, TILE_M), 1)).astype(_bf16)

    def body(i, carry):
        blk = oh_scr[pl.ds(i * TILE_M, TILE_M), :]
        cb = lax.dot_general(tri, blk, (((1,), (0,)), ((), ())),
                             preferred_element_type=_f32)
        c_scr[pl.ds(i * TILE_M, TILE_M), :] = cb + carry
        return carry + jnp.sum(blk.astype(_f32), axis=0, keepdims=True)

    counts = lax.fori_loop(0, T // TILE_M, body, jnp.zeros((1, E), _f32))

    pad = (((counts.astype(_i32) + (TILE_M - 1)) // TILE_M) * TILE_M)
    padf = pad.astype(_f32)
    m8 = (lax.broadcasted_iota(_i32, (E, E), 0)
          < lax.broadcasted_iota(_i32, (E, E), 1)).astype(_f32)
    ps = lax.dot_general(padf, m8, (((1,), (0,)), ((), ())),
                         preferred_element_type=_f32)
    pe = ps + padf

    base_all = ps + c_scr[...]
    d0 = jnp.sum(oh0.astype(_f32) * base_all, axis=1)
    d1 = jnp.sum(oh1.astype(_f32) * base_all, axis=1)
    dest_ref[:, 0:1] = d0[:, None].astype(_i32)
    dest_ref[:, 1:2] = d1[:, None].astype(_i32)

    tstart = 128 * (lax.broadcasted_iota(_i32, (8, 128), 0) * 128
                    + lax.broadcasted_iota(_i32, (8, 128), 1))
    te = jnp.zeros((8, 128), _i32)
    for e in range(E):
        pe_e = lax.slice(pe, (0, e), (1, e + 1)).astype(_i32)
        te += (pe_e <= tstart).astype(_i32)
    te_ref[...] = jnp.minimum(te, E - 1)


def _gate_call(hidden_states, w_gate):
    return pl.pallas_call(
        _gate_body,
        out_shape=[
            jax.ShapeDtypeStruct((T, K), _f32),
            jax.ShapeDtypeStruct((T, K), _i32),
            jax.ShapeDtypeStruct((8, 128), _i32),
        ],
        scratch_shapes=[
            pltpu.VMEM((T, E), _bf16),
            pltpu.VMEM((T, E), _f32),
        ],
    )(hidden_states, w_gate)


def _sc_mesh():
    return plsc.VectorSubcoreMesh(
        core_axis_name="c", subcore_axis_name="s", num_cores=NC)


_DISP_W = T // NW


def _dispatch_call(hs, d0, d1):
    @pl.kernel(
        out_type=jax.ShapeDtypeStruct((P, H), _f32),
        mesh=_sc_mesh(),
        scratch_types=[
            pltpu.VMEM((_DISP_W,), _i32),
            pltpu.VMEM((_DISP_W, H), _f32),
        ],
    )
    def disp(hs_hbm, d0_hbm, d1_hbm, xpad_hbm, idx_v, rows_v):
        wid = lax.axis_index("s") * NC + lax.axis_index("c")
        base = wid * _DISP_W
        for idx_hbm in (d0_hbm, d1_hbm):
            pltpu.sync_copy(hs_hbm.at[pl.ds(base, _DISP_W)], rows_v)
            pltpu.sync_copy(idx_hbm.at[pl.ds(base, _DISP_W)], idx_v)
            pltpu.sync_copy(rows_v, xpad_hbm.at[idx_v])

    return disp(hs, d0, d1)


_GATH_W = 64
_GATH_STEPS = (T * K) // (NW * _GATH_W)


def _gather_call(y_pad, dflat):
    @pl.kernel(
        out_type=jax.ShapeDtypeStruct((T * K, H), _f32),
        mesh=_sc_mesh(),
        scratch_types=[
            pltpu.VMEM((_GATH_W,), _i32),
            pltpu.VMEM((_GATH_W, H), _f32),
        ],
    )
    def gath(ypad_hbm, dflat_hbm, out_hbm, idx_v, rows_v):
        wid = lax.axis_index("s") * NC + lax.axis_index("c")
        for c in range(_GATH_STEPS):
            base = wid * (_GATH_W * _GATH_STEPS) + c * _GATH_W
            pltpu.sync_copy(dflat_hbm.at[pl.ds(base, _GATH_W)], idx_v)
            pltpu.sync_copy(ypad_hbm.at[idx_v], rows_v)
            pltpu.sync_copy(rows_v, out_hbm.at[pl.ds(base, _GATH_W)])

    return gath(y_pad, dflat)


def _ffn_body(eid_ref, x_ref, wg_ref, wu_ref, w2_ref, y_ref):
    x = x_ref[...].astype(_bf16)
    g = lax.dot_general(x, wg_ref[0], (((1,), (0,)), ((), ())),
                        preferred_element_type=_f32)
    u = lax.dot_general(x, wu_ref[0], (((1,), (0,)), ((), ())),
                        preferred_element_type=_f32)
    h = (g * jax.nn.sigmoid(g) * u).astype(_bf16)
    y_ref[...] = lax.dot_general(h, w2_ref[0], (((1,), (0,)), ((), ())),
                                 preferred_element_type=_f32)


def _ffn_call(te, x_pad, wg_b, wu_b, w2_b):
    grid_spec = pltpu.PrefetchScalarGridSpec(
        num_scalar_prefetch=1,
        grid=(NT,),
        in_specs=[
            pl.BlockSpec((TILE_M, H), lambda i, eid: (i, 0)),
            pl.BlockSpec((1, H, DFF), lambda i, eid: (eid[i], 0, 0)),
            pl.BlockSpec((1, H, DFF), lambda i, eid: (eid[i], 0, 0)),
            pl.BlockSpec((1, DFF, H), lambda i, eid: (eid[i], 0, 0)),
        ],
        out_specs=pl.BlockSpec((TILE_M, H), lambda i, eid: (i, 0)),
    )
    return pl.pallas_call(
        _ffn_body,
        grid_spec=grid_spec,
        out_shape=jax.ShapeDtypeStruct((P, H), _f32),
    )(te, x_pad, wg_b, wu_b, w2_b)


def _combine_body(g_ref, s_ref, o_ref):
    o_ref[...] = (g_ref[:, :H] * s_ref[:, 0:1]
                  + g_ref[:, H:] * s_ref[:, 1:2])


def _combine_call(g2, scores):
    return pl.pallas_call(
        _combine_body,
        grid=(T // TILE_M,),
        in_specs=[
            pl.BlockSpec((TILE_M, K * H), lambda i: (i, 0)),
            pl.BlockSpec((TILE_M, K), lambda i: (i, 0)),
        ],
        out_specs=pl.BlockSpec((TILE_M, H), lambda i: (i, 0)),
        out_shape=jax.ShapeDtypeStruct((T, H), _f32),
    )(g2, scores)


def kernel(hidden_states, W_gate, Wg, Wu, W2):
    scores, dest, te8 = _gate_call(hidden_states, W_gate)
    te = te8.reshape(-1)[:NT]
    d0 = dest[:, 0]
    d1 = dest[:, 1]
    dflat = dest.reshape(-1)
    x_pad = _dispatch_call(hidden_states, d0, d1)
    wg_b = Wg.astype(_bf16)
    wu_b = Wu.astype(_bf16)
    w2_b = W2.astype(_bf16)
    y_pad = _ffn_call(te, x_pad, wg_b, wu_b, w2_b)
    g = _gather_call(y_pad, dflat)
    g2 = g.reshape(T, K * H)
    return _combine_call(g2, scores)
